# initial kernel scaffold (unmeasured)
import jax
import jax.numpy as jnp
from jax import lax
from jax.experimental import pallas as pl
from jax.experimental.pallas import tpu as pltpu

N_DEV = 32
N_STAGES = 5
SCALE = 0.08838834764831843


def kernel(x, Wq, Wo, K_ext, V_ext):
    i = lax.axis_index("i")
    x2 = x[0].astype(jnp.bfloat16)
    Wq2 = Wq.astype(jnp.bfloat16)
    Wo2 = Wo.astype(jnp.bfloat16)
    K2 = lax.dynamic_slice_in_dim(K_ext[0], 2 * i, 2, axis=1)
    V2 = lax.dynamic_slice_in_dim(V_ext[0], 2 * i, 2, axis=1)
    K2 = jnp.transpose(K2, (1, 0, 2)).astype(jnp.bfloat16)
    V2 = jnp.transpose(V2, (1, 0, 2)).astype(jnp.bfloat16)

    def body(x_ref, wq_ref, wo_ref, k_ref, v_ref, out_ref,
             recv_bufs, send_sems, recv_sems):
        me = lax.axis_index("i")

        barrier = pltpu.get_barrier_semaphore()
        for s in range(N_STAGES):
            pl.semaphore_signal(
                barrier, inc=1,
                device_id=(me ^ (1 << s),),
                device_id_type=pl.DeviceIdType.MESH,
            )
        pl.semaphore_wait(barrier, N_STAGES)

        Q = jnp.dot(x_ref[...], wq_ref[...],
                    preferred_element_type=jnp.float32) * SCALE
        outs = []
        for h in range(8):
            kv = h // 4
            q_h = Q[:, h * 128:(h + 1) * 128].astype(jnp.bfloat16)
            s_h = lax.dot_general(
                q_h, k_ref[kv], (((1,), (1,)), ((), ())),
                preferred_element_type=jnp.float32)
            m_h = jnp.max(s_h, axis=1, keepdims=True)
            p_h = jnp.exp(s_h - m_h)
            l_h = jnp.sum(p_h, axis=1, keepdims=True)
            o_h = lax.dot_general(
                p_h.astype(jnp.bfloat16), v_ref[kv], (((1,), (0,)), ((), ())),
                preferred_element_type=jnp.float32)
            outs.append((o_h / l_h).astype(jnp.bfloat16))
        A = jnp.concatenate(outs, axis=1)
        out_ref[...] = jnp.dot(A, wo_ref[...],
                               preferred_element_type=jnp.float32)

        for s in range(N_STAGES):
            partner = me ^ (1 << s)
            rdma = pltpu.make_async_remote_copy(
                src_ref=out_ref,
                dst_ref=recv_bufs.at[s],
                send_sem=send_sems.at[s],
                recv_sem=recv_sems.at[s],
                device_id=(partner,),
                device_id_type=pl.DeviceIdType.MESH,
            )
            rdma.start()
            rdma.wait()
            out_ref[...] = out_ref[...] + recv_bufs[s]

    out = pl.pallas_call(
        body,
        out_shape=jax.ShapeDtypeStruct((512, 1024), jnp.float32),
        in_specs=[pl.BlockSpec(memory_space=pltpu.VMEM)] * 5,
        out_specs=pl.BlockSpec(memory_space=pltpu.VMEM),
        scratch_shapes=[
            pltpu.VMEM((N_STAGES, 512, 1024), jnp.float32),
            pltpu.SemaphoreType.DMA((N_STAGES,)),
            pltpu.SemaphoreType.DMA((N_STAGES,)),
        ],
        compiler_params=pltpu.CompilerParams(collective_id=0),
    )(x2, Wq2, Wo2, K2, V2)
    return out.reshape(1, 512, 1024)


# baseline (device time: 214756 ns/iter reference)
import jax
import jax.numpy as jnp
from jax import lax
from jax.experimental import pallas as pl
from jax.experimental.pallas import tpu as pltpu

N_DEV = 32
N_STAGES = 5
SCALE = 0.08838834764831843


def kernel(x, Wq, Wo, K_ext, V_ext):
    i = lax.axis_index("i")
    x2 = x[0].astype(jnp.bfloat16)
    Wq2 = Wq.astype(jnp.bfloat16)
    Wo2 = Wo.astype(jnp.bfloat16)
    K2 = lax.dynamic_slice_in_dim(K_ext[0], 2 * i, 2, axis=1)
    V2 = lax.dynamic_slice_in_dim(V_ext[0], 2 * i, 2, axis=1)
    K2 = jnp.transpose(K2, (1, 0, 2)).astype(jnp.bfloat16)
    V2 = jnp.transpose(V2, (1, 0, 2)).astype(jnp.bfloat16)

    def body(x_ref, wq_ref, wo_ref, k_ref, v_ref, out_ref,
             recv_bufs, send_sems, recv_sems):
        me = lax.axis_index("i")

        barrier = pltpu.get_barrier_semaphore()
        for s in range(N_STAGES):
            pl.semaphore_signal(
                barrier, inc=1,
                device_id=(me ^ (1 << s),),
                device_id_type=pl.DeviceIdType.MESH,
            )
        pl.semaphore_wait(barrier, N_STAGES)

        Q = jnp.dot(x_ref[...], wq_ref[...],
                    preferred_element_type=jnp.float32) * SCALE
        outs = []
        for h in range(8):
            kv = h // 4
            q_h = Q[:, h * 128:(h + 1) * 128].astype(jnp.bfloat16)
            s_h = lax.dot_general(
                q_h, k_ref[kv], (((1,), (1,)), ((), ())),
                preferred_element_type=jnp.float32)
            m_h = jnp.max(s_h, axis=1, keepdims=True)
            p_h = jnp.exp(s_h - m_h)
            l_h = jnp.sum(p_h, axis=1, keepdims=True)
            o_h = lax.dot_general(
                p_h.astype(jnp.bfloat16), v_ref[kv], (((1,), (0,)), ((), ())),
                preferred_element_type=jnp.float32)
            outs.append((o_h / l_h).astype(jnp.bfloat16))
        A = jnp.concatenate(outs, axis=1)
        out_ref[...] = jnp.dot(A, wo_ref[...],
                               preferred_element_type=jnp.float32)

        for s in range(N_STAGES):
            partner = me ^ (1 << s)
            rdma = pltpu.make_async_remote_copy(
                src_ref=out_ref,
                dst_ref=recv_bufs.at[s],
                send_sem=send_sems.at[s],
                recv_sem=recv_sems.at[s],
                device_id=(partner,),
                device_id_type=pl.DeviceIdType.MESH,
            )
            rdma.start()
            rdma.wait()
            out_ref[...] = out_ref[...] + recv_bufs[s]

    out = pl.pallas_call(
        body,
        out_shape=jax.ShapeDtypeStruct((512, 1024), jnp.float32),
        in_specs=[pl.BlockSpec(memory_space=pltpu.VMEM)] * 5,
        out_specs=pl.BlockSpec(memory_space=pltpu.VMEM),
        scratch_shapes=[
            pltpu.VMEM((N_STAGES, 512, 1024), jnp.float32),
            pltpu.SemaphoreType.DMA((N_STAGES,)),
            pltpu.SemaphoreType.DMA((N_STAGES,)),
        ],
        compiler_params=pltpu.CompilerParams(
            collective_id=0, vmem_limit_bytes=100 * 1024 * 1024),
    )(x2, Wq2, Wo2, K2, V2)
    return out.reshape(1, 512, 1024)


# device time: 88958 ns/iter; 2.4141x vs baseline; 2.4141x over previous
import jax
import jax.numpy as jnp
from jax import lax
from jax.experimental import pallas as pl
from jax.experimental.pallas import tpu as pltpu

N_DEV = 32
N_STAGES = 5
SCALE = 0.08838834764831843

RS_OFF = {4: 0, 3: 256, 2: 384, 1: 448, 0: 480}
AG_OFF = {0: 496, 1: 512, 2: 544, 3: 608, 4: 736}


def kernel(x, Wq, Wo, K_ext, V_ext):
    i = lax.axis_index("i")
    x2 = x[0].astype(jnp.bfloat16)
    Wq2 = Wq.astype(jnp.bfloat16)
    Wo2 = Wo.astype(jnp.bfloat16)
    K2 = lax.dynamic_slice_in_dim(K_ext[0], 2 * i, 2, axis=1)
    V2 = lax.dynamic_slice_in_dim(V_ext[0], 2 * i, 2, axis=1)
    K2 = jnp.transpose(K2, (1, 0, 2)).astype(jnp.bfloat16)
    V2 = jnp.transpose(V2, (1, 0, 2)).astype(jnp.bfloat16)

    def body(x_ref, wq_ref, wo_ref, k_ref, v_ref, out_ref,
             comm_ref, send_ref, send_sems, recv_sems):
        me = lax.axis_index("i")

        barrier = pltpu.get_barrier_semaphore()
        for s in range(N_STAGES):
            pl.semaphore_signal(
                barrier, inc=1,
                device_id=(me ^ (1 << s),),
                device_id_type=pl.DeviceIdType.MESH,
            )
        pl.semaphore_wait(barrier, N_STAGES)

        Q = jnp.dot(x_ref[...], wq_ref[...],
                    preferred_element_type=jnp.float32) * SCALE
        outs = []
        for h in range(8):
            kv = h // 4
            q_h = Q[:, h * 128:(h + 1) * 128].astype(jnp.bfloat16)
            s_h = lax.dot_general(
                q_h, k_ref[kv], (((1,), (1,)), ((), ())),
                preferred_element_type=jnp.float32)
            m_h = jnp.max(s_h, axis=1, keepdims=True)
            p_h = jnp.exp(s_h - m_h)
            l_h = jnp.sum(p_h, axis=1, keepdims=True)
            o_h = lax.dot_general(
                p_h.astype(jnp.bfloat16), v_ref[kv], (((1,), (0,)), ((), ())),
                preferred_element_type=jnp.float32)
            outs.append((o_h / l_h).astype(jnp.bfloat16))
        A = jnp.concatenate(outs, axis=1)
        out_ref[...] = jnp.dot(A, wo_ref[...],
                               preferred_element_type=jnp.float32)

        base = me - me
        for k in range(4, -1, -1):
            dev_bit = 1 << (4 - k)
            partner = me ^ dev_bit
            myb = (me >> (4 - k)) & 1
            sz = 1 << k
            w = 16 * sz
            lo_keep = (base + myb * sz) * 16
            lo_send = (base + (1 - myb) * sz) * 16
            sem = 4 - k
            send_ref[pl.ds(RS_OFF[k], w), :] = (
                out_ref[pl.ds(lo_send, w), :].astype(jnp.bfloat16))
            rdma = pltpu.make_async_remote_copy(
                src_ref=send_ref.at[pl.ds(RS_OFF[k], w), :],
                dst_ref=comm_ref.at[pl.ds(RS_OFF[k], w), :],
                send_sem=send_sems.at[sem],
                recv_sem=recv_sems.at[sem],
                device_id=(partner,),
                device_id_type=pl.DeviceIdType.MESH,
            )
            rdma.start()
            rdma.wait()
            out_ref[pl.ds(lo_keep, w), :] = (
                out_ref[pl.ds(lo_keep, w), :]
                + comm_ref[pl.ds(RS_OFF[k], w), :].astype(jnp.float32))
            base = base + myb * sz

        cur_lo = base * 16
        for k in range(5):
            dev_bit = 1 << (4 - k)
            partner = me ^ dev_bit
            myb = (me >> (4 - k)) & 1
            w = 16 << k
            sem = 5 + k
            send_ref[pl.ds(AG_OFF[k], w), :] = (
                out_ref[pl.ds(cur_lo, w), :].astype(jnp.bfloat16))
            rdma = pltpu.make_async_remote_copy(
                src_ref=send_ref.at[pl.ds(AG_OFF[k], w), :],
                dst_ref=comm_ref.at[pl.ds(AG_OFF[k], w), :],
                send_sem=send_sems.at[sem],
                recv_sem=recv_sems.at[sem],
                device_id=(partner,),
                device_id_type=pl.DeviceIdType.MESH,
            )
            rdma.start()
            rdma.wait()
            partner_lo = cur_lo + (1 - 2 * myb) * w
            out_ref[pl.ds(partner_lo, w), :] = (
                comm_ref[pl.ds(AG_OFF[k], w), :].astype(jnp.float32))
            cur_lo = cur_lo - myb * w

    out = pl.pallas_call(
        body,
        out_shape=jax.ShapeDtypeStruct((512, 1024), jnp.float32),
        in_specs=[pl.BlockSpec(memory_space=pltpu.VMEM)] * 5,
        out_specs=pl.BlockSpec(memory_space=pltpu.VMEM),
        scratch_shapes=[
            pltpu.VMEM((992, 1024), jnp.bfloat16),
            pltpu.VMEM((992, 1024), jnp.bfloat16),
            pltpu.SemaphoreType.DMA((10,)),
            pltpu.SemaphoreType.DMA((10,)),
        ],
        compiler_params=pltpu.CompilerParams(
            collective_id=0, vmem_limit_bytes=100 * 1024 * 1024),
    )(x2, Wq2, Wo2, K2, V2)
    return out.reshape(1, 512, 1024)


# device time: 73202 ns/iter; 2.9337x vs baseline; 1.2152x over previous
import jax
import jax.numpy as jnp
from jax import lax
from jax.experimental import pallas as pl
from jax.experimental.pallas import tpu as pltpu

N_DEV = 32
N_STAGES = 5
SCALE = 0.08838834764831843

RS_OFF = {4: 0, 3: 256, 2: 384, 1: 448, 0: 480}
AG_OFF = {0: 496, 1: 512, 2: 544, 3: 608, 4: 736}


def kernel(x, Wq, Wo, K_ext, V_ext):
    x2 = x[0]
    K3 = K_ext[0]
    V3 = V_ext[0]

    def body(x_ref, wq_ref, wo_ref, kext_ref, vext_ref, out_ref,
             comm_ref, send_ref, kv_vmem, send_sems, recv_sems, copy_sems):
        me = lax.axis_index("i")

        kcopy = pltpu.make_async_copy(
            kext_ref.at[:, pl.ds(2 * me, 2), :], kv_vmem.at[0],
            copy_sems.at[0])
        vcopy = pltpu.make_async_copy(
            vext_ref.at[:, pl.ds(2 * me, 2), :], kv_vmem.at[1],
            copy_sems.at[1])
        kcopy.start()
        vcopy.start()

        barrier = pltpu.get_barrier_semaphore()
        for s in range(N_STAGES):
            pl.semaphore_signal(
                barrier, inc=1,
                device_id=(me ^ (1 << s),),
                device_id_type=pl.DeviceIdType.MESH,
            )
        pl.semaphore_wait(barrier, N_STAGES)

        Q = jnp.dot(x_ref[...].astype(jnp.bfloat16),
                    wq_ref[...].astype(jnp.bfloat16),
                    preferred_element_type=jnp.float32) * SCALE
        kcopy.wait()
        vcopy.wait()
        outs = []
        for h in range(8):
            kv = h // 4
            q_h = Q[:, h * 128:(h + 1) * 128].astype(jnp.bfloat16)
            k_h = kv_vmem[0, :, kv, :].astype(jnp.bfloat16)
            v_h = kv_vmem[1, :, kv, :].astype(jnp.bfloat16)
            s_h = lax.dot_general(
                q_h, k_h, (((1,), (1,)), ((), ())),
                preferred_element_type=jnp.float32)
            m_h = jnp.max(s_h, axis=1, keepdims=True)
            p_h = jnp.exp(s_h - m_h)
            l_h = jnp.sum(p_h, axis=1, keepdims=True)
            o_h = lax.dot_general(
                p_h.astype(jnp.bfloat16), v_h, (((1,), (0,)), ((), ())),
                preferred_element_type=jnp.float32)
            outs.append((o_h / l_h).astype(jnp.bfloat16))
        A = jnp.concatenate(outs, axis=1)
        out_ref[...] = jnp.dot(A, wo_ref[...].astype(jnp.bfloat16),
                               preferred_element_type=jnp.float32)

        base = me - me
        for k in range(4, -1, -1):
            dev_bit = 1 << (4 - k)
            partner = me ^ dev_bit
            myb = (me >> (4 - k)) & 1
            sz = 1 << k
            w = 16 * sz
            lo_keep = (base + myb * sz) * 16
            lo_send = (base + (1 - myb) * sz) * 16
            sem = 4 - k
            send_ref[pl.ds(RS_OFF[k], w), :] = (
                out_ref[pl.ds(lo_send, w), :].astype(jnp.bfloat16))
            rdma = pltpu.make_async_remote_copy(
                src_ref=send_ref.at[pl.ds(RS_OFF[k], w), :],
                dst_ref=comm_ref.at[pl.ds(RS_OFF[k], w), :],
                send_sem=send_sems.at[sem],
                recv_sem=recv_sems.at[sem],
                device_id=(partner,),
                device_id_type=pl.DeviceIdType.MESH,
            )
            rdma.start()
            rdma.wait()
            out_ref[pl.ds(lo_keep, w), :] = (
                out_ref[pl.ds(lo_keep, w), :]
                + comm_ref[pl.ds(RS_OFF[k], w), :].astype(jnp.float32))
            base = base + myb * sz

        cur_lo = base * 16
        for k in range(5):
            dev_bit = 1 << (4 - k)
            partner = me ^ dev_bit
            myb = (me >> (4 - k)) & 1
            w = 16 << k
            sem = 5 + k
            send_ref[pl.ds(AG_OFF[k], w), :] = (
                out_ref[pl.ds(cur_lo, w), :].astype(jnp.bfloat16))
            rdma = pltpu.make_async_remote_copy(
                src_ref=send_ref.at[pl.ds(AG_OFF[k], w), :],
                dst_ref=comm_ref.at[pl.ds(AG_OFF[k], w), :],
                send_sem=send_sems.at[sem],
                recv_sem=recv_sems.at[sem],
                device_id=(partner,),
                device_id_type=pl.DeviceIdType.MESH,
            )
            rdma.start()
            rdma.wait()
            partner_lo = cur_lo + (1 - 2 * myb) * w
            out_ref[pl.ds(partner_lo, w), :] = (
                comm_ref[pl.ds(AG_OFF[k], w), :].astype(jnp.float32))
            cur_lo = cur_lo - myb * w

    out = pl.pallas_call(
        body,
        out_shape=jax.ShapeDtypeStruct((512, 1024), jnp.float32),
        in_specs=[pl.BlockSpec(memory_space=pltpu.VMEM)] * 3
        + [pl.BlockSpec(memory_space=pltpu.MemorySpace.HBM)] * 2,
        out_specs=pl.BlockSpec(memory_space=pltpu.VMEM),
        scratch_shapes=[
            pltpu.VMEM((992, 1024), jnp.bfloat16),
            pltpu.VMEM((992, 1024), jnp.bfloat16),
            pltpu.VMEM((2, 2048, 2, 128), jnp.float32),
            pltpu.SemaphoreType.DMA((10,)),
            pltpu.SemaphoreType.DMA((10,)),
            pltpu.SemaphoreType.DMA((2,)),
        ],
        compiler_params=pltpu.CompilerParams(
            collective_id=0, vmem_limit_bytes=100 * 1024 * 1024),
    )(x2, Wq, Wo, K3, V3)
    return out.reshape(1, 512, 1024)
